# baseline (speedup 1.0000x reference)
import jax
import jax.numpy as jnp
from jax.experimental import pallas as pl

_MARGIN = 1.0


def _loss_kernel(wo_ref, lab_ref, tab_ref, out_ref):
    B = wo_ref.shape[0]
    C = tab_ref.shape[0]
    woT = wo_ref[:].T
    x2 = jnp.sum(woT * woT, axis=0, keepdims=True)
    inv = jax.lax.rsqrt(jnp.maximum(x2, 1e-24))
    wnT = woT * (-2.0 * inv)
    xn2 = x2 * (inv * inv)

    tab = tab_ref[:]
    t2 = jnp.sum(tab * tab, axis=1, keepdims=True)
    dots = jnp.dot(tab, wnT, preferred_element_type=jnp.float32)
    d2 = (xn2 + t2) + dots

    lab = lab_ref[:].reshape(1, B)
    rows = jax.lax.broadcasted_iota(jnp.int32, (C, B), 0)
    is_lab = rows == lab
    lab_d2 = jnp.sum(jnp.where(is_lab, d2, 0.0), axis=0, keepdims=True)
    min_d2 = jnp.min(jnp.where(is_lab, jnp.inf, d2), axis=0, keepdims=True)
    lab_d = lab_d2 * jax.lax.rsqrt(jnp.maximum(lab_d2, 1e-30))
    min_d = min_d2 * jax.lax.rsqrt(jnp.maximum(min_d2, 1e-30))
    s = jnp.sum(lab_d - min_d, axis=1, keepdims=True)
    out_ref[:, :] = _MARGIN + s / B


def kernel(WO, label, table):
    out = pl.pallas_call(
        _loss_kernel,
        out_shape=jax.ShapeDtypeStruct((1, 1), jnp.float32),
    )(WO, label.astype(jnp.int32), table)
    return out[0, 0]


# speedup vs baseline: 1.3298x; 1.3298x over previous
import jax
import jax.numpy as jnp
from jax.experimental import pallas as pl

_MARGIN = 1.0


def _loss_kernel(woT_ref, lab_ref, tab_ref, out_ref):
    B = woT_ref.shape[1]
    C = tab_ref.shape[0]
    woT = woT_ref[:]
    x2 = jnp.sum(woT * woT, axis=0, keepdims=True)
    inv = jax.lax.rsqrt(jnp.maximum(x2, 1e-24))
    wnT = woT * (-2.0 * inv)
    xn2 = x2 * (inv * inv)
    rhs = jnp.concatenate([wnT, jnp.ones((1, B), jnp.float32)], axis=0)

    tab = tab_ref[:]
    t2 = jnp.sum(tab * tab, axis=1, keepdims=True)
    lhs = jnp.concatenate([tab, t2], axis=1)
    d2p = jnp.dot(lhs, rhs, preferred_element_type=jnp.float32)

    lab = lab_ref[:]
    rows = jax.lax.broadcasted_iota(jnp.int32, (C, B), 0)
    is_lab = rows == lab
    lab_d2 = jnp.sum(jnp.where(is_lab, d2p, 0.0), axis=0, keepdims=True) + xn2
    min_d2 = jnp.min(jnp.where(is_lab, jnp.inf, d2p), axis=0, keepdims=True) + xn2
    lab_d = lab_d2 * jax.lax.rsqrt(jnp.maximum(lab_d2, 1e-30))
    min_d = min_d2 * jax.lax.rsqrt(jnp.maximum(min_d2, 1e-30))
    s = jnp.sum(lab_d - min_d, axis=1, keepdims=True)
    out_ref[:, :] = _MARGIN + s / B


def kernel(WO, label, table):
    B, _ = WO.shape
    out = pl.pallas_call(
        _loss_kernel,
        out_shape=jax.ShapeDtypeStruct((1, 1), jnp.float32),
    )(WO.T, label.astype(jnp.int32).reshape(1, B), table)
    return out[0, 0]
